# Initial kernel scaffold; baseline (speedup 1.0000x reference)
import jax
import jax.numpy as jnp
from jax.experimental import pallas as pl


def kernel(taxPayer_feats, person_feats, item_feats, trans_adj_list, pattern_name, P_company, P_person, P_item, W_CC):
    raise NotImplementedError("write your pallas kernel here")


# speedup vs baseline: 2.9390x; 2.9390x over previous
import functools

import jax
import jax.numpy as jnp
from jax import lax
from jax.experimental import pallas as pl
from jax.experimental.pallas import tpu as pltpu
from jax.experimental.pallas import tpu_sc as plsc

N = 10000
E = 320000
D = 128
ALPHA = 0.2

NC = 2
NS = 16
NW = NC * NS
EW = E // NW
CB = 80
K = EW // CB
LANES = 16
CPR = D // LANES


def _tc_proj_body(x_ref, p_ref, w_ref, a_ref, b_ref):
    f = jnp.dot(x_ref[...], p_ref[...],
                preferred_element_type=jnp.float32,
                precision=lax.Precision.HIGHEST)
    w = w_ref[...]
    a_ref[...] = jnp.dot(f, w[:D], preferred_element_type=jnp.float32,
                         precision=lax.Precision.HIGHEST)
    b_ref[...] = jnp.dot(f, w[D:], preferred_element_type=jnp.float32,
                         precision=lax.Precision.HIGHEST)


def _tc_project(x, p, w):
    return pl.pallas_call(
        _tc_proj_body,
        out_shape=(jax.ShapeDtypeStruct((N, D), jnp.float32),
                   jax.ShapeDtypeStruct((N, D), jnp.float32)),
    )(x, p, w)


def _sc_edge_body(a_hbm, b_hbm, idx0_hbm, idx1_hbm, out_hbm,
                  idx0_v, idx1_v, bufA, bufB, bufO, semA, semB, semO):
    c = lax.axis_index("c")
    s = lax.axis_index("s")
    wid = s * NC + c

    pltpu.sync_copy(idx0_hbm.at[wid], idx0_v)
    pltpu.sync_copy(idx1_hbm.at[wid], idx1_v)

    row_base = wid * EW

    def start_gathers(k, slot):
        cpA = pltpu.async_copy(a_hbm.at[idx0_v.at[k]], bufA.at[slot], semA)
        cpB = pltpu.async_copy(b_hbm.at[idx1_v.at[k]], bufB.at[slot], semB)
        return cpA, cpB

    start_gathers(0, 0)

    def chunk_body(k, carry):
        slot = lax.rem(k, 2)
        nslot = 1 - slot
        pltpu.make_async_copy(a_hbm.at[idx0_v.at[k]], bufA.at[slot], semA).wait()
        pltpu.make_async_copy(b_hbm.at[idx1_v.at[k]], bufB.at[slot], semB).wait()

        @pl.when(k + 1 < K)
        def _():
            start_gathers(k + 1, nslot)

        @pl.when(k >= 2)
        def _():
            pltpu.make_async_copy(
                bufO.at[slot], out_hbm.at[pl.ds(row_base + (k - 2) * CB, CB)],
                semO).wait()

        def row_body(r, acc):
            for cc in range(CPR):
                av = bufA[slot, r, pl.ds(cc * LANES, LANES)]
                bv = bufB[slot, r, pl.ds(cc * LANES, LANES)]
                sv = av + bv
                bufO[slot, r, pl.ds(cc * LANES, LANES)] = jnp.maximum(
                    sv, sv * jnp.float32(ALPHA))
            return acc
        lax.fori_loop(0, CB, row_body, 0, unroll=2)

        pltpu.async_copy(bufO.at[slot],
                         out_hbm.at[pl.ds(row_base + k * CB, CB)], semO)
        return carry

    lax.fori_loop(0, K, chunk_body, 0)

    pltpu.make_async_copy(
        bufO.at[(K - 2) % 2],
        out_hbm.at[pl.ds(row_base + (K - 2) * CB, CB)], semO).wait()
    pltpu.make_async_copy(
        bufO.at[(K - 1) % 2],
        out_hbm.at[pl.ds(row_base + (K - 1) * CB, CB)], semO).wait()


@jax.jit
def _sc_edge(a, b, idx0, idx1):
    mesh = plsc.VectorSubcoreMesh(core_axis_name="c", subcore_axis_name="s")
    return pl.kernel(
        _sc_edge_body,
        out_type=jax.ShapeDtypeStruct((E, D), jnp.float32),
        mesh=mesh,
        scratch_types=[
            pltpu.VMEM((K, CB), jnp.int32),
            pltpu.VMEM((K, CB), jnp.int32),
            pltpu.VMEM((2, CB, D), jnp.float32),
            pltpu.VMEM((2, CB, D), jnp.float32),
            pltpu.VMEM((2, CB, D), jnp.float32),
            pltpu.SemaphoreType.DMA,
            pltpu.SemaphoreType.DMA,
            pltpu.SemaphoreType.DMA,
        ],
    )(a, b, idx0, idx1)


def kernel(taxPayer_feats, person_feats, item_feats, trans_adj_list,
           pattern_name, P_company, P_person, P_item, W_CC):
    a, b = _tc_project(taxPayer_feats, P_company, W_CC)
    idx0 = trans_adj_list[0].astype(jnp.int32).reshape(NW, K, CB)
    idx1 = trans_adj_list[1].astype(jnp.int32).reshape(NW, K, CB)
    return _sc_edge(a, b, idx0, idx1)


# speedup vs baseline: 7.3515x; 2.5013x over previous
import jax
import jax.numpy as jnp
from jax import lax
from jax.experimental import pallas as pl
from jax.experimental.pallas import tpu as pltpu
from jax.experimental.pallas import tpu_sc as plsc

N = 10000
E = 320000
D = 128
ALPHA = 0.2

NC = 2
NS = 16
NW = NC * NS
EW = E // NW
CB = 40
K = EW // CB
NBUF = 5
KO = K // NBUF
LANES = 16
CPR = D // LANES


def _tc_proj_body(x_ref, p_ref, w_ref, a_ref, b_ref):
    f = jnp.dot(x_ref[...], p_ref[...],
                preferred_element_type=jnp.float32,
                precision=lax.Precision.HIGHEST)
    w = w_ref[...]
    a_ref[...] = jnp.dot(f, w[:D], preferred_element_type=jnp.float32,
                         precision=lax.Precision.HIGHEST)
    b_ref[...] = jnp.dot(f, w[D:], preferred_element_type=jnp.float32,
                         precision=lax.Precision.HIGHEST)


def _tc_project(x, p, w):
    return pl.pallas_call(
        _tc_proj_body,
        out_shape=(jax.ShapeDtypeStruct((N, D), jnp.float32),
                   jax.ShapeDtypeStruct((N, D), jnp.float32)),
    )(x, p, w)


def _sc_edge_body(a_hbm, b_hbm, idx0_hbm, idx1_hbm, out_hbm,
                  idx0_v, idx1_v, bufA, bufB, semA, semB, semO):
    c = lax.axis_index("c")
    s = lax.axis_index("s")
    wid = s * NC + c

    pltpu.sync_copy(idx0_hbm.at[wid], idx0_v)
    pltpu.sync_copy(idx1_hbm.at[wid], idx1_v)

    row_base = wid * EW

    def gather_issue(k, slot):
        pltpu.async_copy(a_hbm.at[idx0_v.at[k]], bufA.at[slot], semA)
        pltpu.async_copy(b_hbm.at[idx1_v.at[k]], bufB.at[slot], semB)

    def gather_wait(k, slot):
        pltpu.make_async_copy(a_hbm.at[idx0_v.at[k]], bufA.at[slot],
                              semA).wait()
        pltpu.make_async_copy(b_hbm.at[idx1_v.at[k]], bufB.at[slot],
                              semB).wait()

    def scatter_issue(k, slot):
        pltpu.async_copy(bufA.at[slot],
                         out_hbm.at[pl.ds(row_base + k * CB, CB)], semO)

    def scatter_drain_one(k, slot):
        pltpu.make_async_copy(bufA.at[slot],
                              out_hbm.at[pl.ds(row_base + k * CB, CB)],
                              semO).wait()

    def compute_inplace(slot):
        @plsc.parallel_loop(0, CB, 1, unroll=4)
        def _(r):
            for cc in range(CPR):
                dsl = pl.ds(cc * LANES, LANES)
                av = bufA[slot, r, dsl]
                bv = bufB[slot, r, dsl]
                sv = av + bv
                bufA[slot, r, dsl] = jnp.maximum(sv, sv * jnp.float32(ALPHA))

    def step(k, b, do_drain, next_k_ok):
        gather_wait(k, b)
        compute_inplace(b)
        scatter_issue(k, b)
        if do_drain:
            scatter_drain_one(k, b)
        if next_k_ok:
            gather_issue(k + (NBUF - 2), (b + (NBUF - 2)) % NBUF)

    for kp in range(NBUF - 2):
        gather_issue(kp, kp)

    for b in range(NBUF):
        step(b, b, do_drain=(b >= 2), next_k_ok=True)

    def outer(k5, carry):
        k0 = k5 * NBUF
        for b in range(NBUF):
            step(k0 + b, b, do_drain=True, next_k_ok=True)
        return carry
    lax.fori_loop(1, KO - 1, outer, 0)

    for b in range(NBUF):
        step(K - NBUF + b, b, do_drain=True,
             next_k_ok=(K - NBUF + b + NBUF - 2 < K))

    scatter_drain_one(K - 2, (K - 2) % NBUF)
    scatter_drain_one(K - 1, (K - 1) % NBUF)


@jax.jit
def _sc_edge(a, b, idx0, idx1):
    mesh = plsc.VectorSubcoreMesh(core_axis_name="c", subcore_axis_name="s")
    return pl.kernel(
        _sc_edge_body,
        out_type=jax.ShapeDtypeStruct((E, D), jnp.float32),
        mesh=mesh,
        scratch_types=[
            pltpu.VMEM((K, CB), jnp.int32),
            pltpu.VMEM((K, CB), jnp.int32),
            pltpu.VMEM((NBUF, CB, D), jnp.float32),
            pltpu.VMEM((NBUF, CB, D), jnp.float32),
            pltpu.SemaphoreType.DMA,
            pltpu.SemaphoreType.DMA,
            pltpu.SemaphoreType.DMA,
        ],
    )(a, b, idx0, idx1)


def kernel(taxPayer_feats, person_feats, item_feats, trans_adj_list,
           pattern_name, P_company, P_person, P_item, W_CC):
    a, b = _tc_project(taxPayer_feats, P_company, W_CC)
    idx0 = trans_adj_list[0].astype(jnp.int32).reshape(NW, K, CB)
    idx1 = trans_adj_list[1].astype(jnp.int32).reshape(NW, K, CB)
    return _sc_edge(a, b, idx0, idx1)


# speedup vs baseline: 7.6605x; 1.0420x over previous
import jax
import jax.numpy as jnp
from jax import lax
from jax.experimental import pallas as pl
from jax.experimental.pallas import tpu as pltpu
from jax.experimental.pallas import tpu_sc as plsc

N = 10000
E = 320000
D = 128
ALPHA = 0.2

NC = 2
NS = 16
NW = NC * NS
EW = E // NW
CB = 40
K = EW // CB
NBUF = 5
KO = K // NBUF
LANES = 16
CPR = D // LANES


def _tc_proj_body(x_ref, p_ref, w_ref, a_ref, b_ref):
    f = jnp.dot(x_ref[...], p_ref[...],
                preferred_element_type=jnp.float32)
    w = w_ref[...]
    a_ref[...] = jnp.dot(f, w[:D], preferred_element_type=jnp.float32)
    b_ref[...] = jnp.dot(f, w[D:], preferred_element_type=jnp.float32)


def _tc_project(x, p, w):
    return pl.pallas_call(
        _tc_proj_body,
        out_shape=(jax.ShapeDtypeStruct((N, D), jnp.float32),
                   jax.ShapeDtypeStruct((N, D), jnp.float32)),
    )(x, p, w)


def _sc_edge_body(a_hbm, b_hbm, idx0_hbm, idx1_hbm, out_hbm,
                  idx0_v, idx1_v, bufA, bufB, semA, semB, semO):
    c = lax.axis_index("c")
    s = lax.axis_index("s")
    wid = s * NC + c

    pltpu.sync_copy(idx0_hbm.at[wid], idx0_v)
    pltpu.sync_copy(idx1_hbm.at[wid], idx1_v)

    row_base = wid * EW

    def gather_issue(k, slot):
        pltpu.async_copy(a_hbm.at[idx0_v.at[k]], bufA.at[slot], semA)
        pltpu.async_copy(b_hbm.at[idx1_v.at[k]], bufB.at[slot], semB)

    def gather_wait(k, slot):
        pltpu.make_async_copy(a_hbm.at[idx0_v.at[k]], bufA.at[slot],
                              semA).wait()
        pltpu.make_async_copy(b_hbm.at[idx1_v.at[k]], bufB.at[slot],
                              semB).wait()

    def scatter_issue(k, slot):
        pltpu.async_copy(bufA.at[slot],
                         out_hbm.at[pl.ds(row_base + k * CB, CB)], semO)

    def scatter_drain_one(k, slot):
        pltpu.make_async_copy(bufA.at[slot],
                              out_hbm.at[pl.ds(row_base + k * CB, CB)],
                              semO).wait()

    def compute_inplace(slot):
        @plsc.parallel_loop(0, CB, 1, unroll=8)
        def _(r):
            for cc in range(CPR):
                dsl = pl.ds(cc * LANES, LANES)
                av = bufA[slot, r, dsl]
                bv = bufB[slot, r, dsl]
                sv = av + bv
                bufA[slot, r, dsl] = jnp.maximum(sv, sv * jnp.float32(ALPHA))

    def step(k, b, do_drain, next_k_ok):
        gather_wait(k, b)
        compute_inplace(b)
        scatter_issue(k, b)
        if do_drain:
            scatter_drain_one(k, b)
        if next_k_ok:
            gather_issue(k + (NBUF - 2), (b + (NBUF - 2)) % NBUF)

    for kp in range(NBUF - 2):
        gather_issue(kp, kp)

    for b in range(NBUF):
        step(b, b, do_drain=(b >= 2), next_k_ok=True)

    def outer(k5, carry):
        k0 = k5 * NBUF
        for b in range(NBUF):
            step(k0 + b, b, do_drain=True, next_k_ok=True)
        return carry
    lax.fori_loop(1, KO - 1, outer, 0)

    for b in range(NBUF):
        step(K - NBUF + b, b, do_drain=True,
             next_k_ok=(K - NBUF + b + NBUF - 2 < K))

    scatter_drain_one(K - 2, (K - 2) % NBUF)
    scatter_drain_one(K - 1, (K - 1) % NBUF)


@jax.jit
def _sc_edge(a, b, idx0, idx1):
    mesh = plsc.VectorSubcoreMesh(core_axis_name="c", subcore_axis_name="s")
    return pl.kernel(
        _sc_edge_body,
        out_type=jax.ShapeDtypeStruct((E, D), jnp.float32),
        mesh=mesh,
        scratch_types=[
            pltpu.VMEM((K, CB), jnp.int32),
            pltpu.VMEM((K, CB), jnp.int32),
            pltpu.VMEM((NBUF, CB, D), jnp.float32),
            pltpu.VMEM((NBUF, CB, D), jnp.float32),
            pltpu.SemaphoreType.DMA,
            pltpu.SemaphoreType.DMA,
            pltpu.SemaphoreType.DMA,
        ],
    )(a, b, idx0, idx1)


def kernel(taxPayer_feats, person_feats, item_feats, trans_adj_list,
           pattern_name, P_company, P_person, P_item, W_CC):
    a, b = _tc_project(taxPayer_feats, P_company, W_CC)
    idx0 = trans_adj_list[0].astype(jnp.int32).reshape(NW, K, CB)
    idx1 = trans_adj_list[1].astype(jnp.int32).reshape(NW, K, CB)
    return _sc_edge(a, b, idx0, idx1)
